# Initial kernel scaffold; baseline (speedup 1.0000x reference)
import jax
import jax.numpy as jnp
from jax.experimental import pallas as pl


def kernel(x, emb_table, W, b, bn_gamma, bn_beta, ln_gamma, ln_beta):
    raise NotImplementedError("write your pallas kernel here")


# speedup vs baseline: 6.8764x; 6.8764x over previous
import functools

import jax
import jax.numpy as jnp
from jax import lax
from jax.experimental import pallas as pl
from jax.experimental.pallas import tpu as pltpu
from jax.experimental.pallas import tpu_sc as plsc

VOCAB = 100000
D = 128
B = 4096
L = 50
EPS = 1e-5

NC = 2
NS = 16
NW = NC * NS
ROWS_PER_W = B // NW
CB = 2
CHUNK_IDX = CB * L
CHUNKS = ROWS_PER_W // CB
NV = D // 16


def _pool_sc(x3, emb_table):
    @functools.partial(
        pl.kernel,
        out_type=jax.ShapeDtypeStruct((B, D), jnp.float32),
        mesh=plsc.VectorSubcoreMesh(core_axis_name="c", subcore_axis_name="s"),
        scratch_types=[
            pltpu.VMEM((CHUNKS, CHUNK_IDX), jnp.int32),
            pltpu.VMEM((CHUNK_IDX, D), jnp.float32),
            pltpu.VMEM((ROWS_PER_W, D), jnp.float32),
            pltpu.SemaphoreType.DMA,
        ],
    )
    def k(x_hbm, table_hbm, out_hbm, idx_v, rows_v, pooled_v, sem):
        wid = lax.axis_index("s") * NC + lax.axis_index("c")
        pltpu.sync_copy(x_hbm.at[wid], idx_v)

        def chunk_body(c, carry):
            pltpu.async_copy(table_hbm.at[idx_v.at[c]], rows_v, sem).wait()
            for b_local in range(CB):
                def red_body(j, acc):
                    row = b_local * L + j
                    return tuple(
                        acc[k] + rows_v[row, pl.ds(k * 16, 16)]
                        for k in range(NV)
                    )
                acc0 = tuple(jnp.zeros((16,), jnp.float32) for _ in range(NV))
                acc = lax.fori_loop(0, L, red_body, acc0)
                for k in range(NV):
                    pooled_v[c * CB + b_local, pl.ds(k * 16, 16)] = acc[k]
            return carry

        lax.fori_loop(0, CHUNKS, chunk_body, 0)
        pltpu.sync_copy(pooled_v, out_hbm.at[pl.ds(wid * ROWS_PER_W, ROWS_PER_W)])

    return k(x3, emb_table)


def _dense_kernel(pooled_ref, w_ref, b_ref, bng_ref, bnb_ref, lng_ref, lnb_ref,
                  out_ref):
    p = pooled_ref[...] * (1.0 / L)
    h = lax.dot_general(
        p, w_ref[...], (((1,), (1,)), ((), ())),
        preferred_element_type=jnp.float32,
        precision=lax.Precision.HIGHEST,
    ) + b_ref[...]
    mu = jnp.mean(h, axis=0, keepdims=True)
    var = jnp.mean((h - mu) * (h - mu), axis=0, keepdims=True)
    h = (h - mu) * lax.rsqrt(var + EPS) * bng_ref[...] + bnb_ref[...]
    m = jnp.mean(h, axis=1, keepdims=True)
    v = jnp.mean((h - m) * (h - m), axis=1, keepdims=True)
    out_ref[...] = (h - m) * lax.rsqrt(v + EPS) * lng_ref[...] + lnb_ref[...]


def _dense_tc(pooled, W, b, bn_gamma, bn_beta, ln_gamma, ln_beta):
    return pl.pallas_call(
        _dense_kernel,
        out_shape=jax.ShapeDtypeStruct((B, D), jnp.float32),
    )(pooled, W, b.reshape(1, D), bn_gamma.reshape(1, D),
      bn_beta.reshape(1, D), ln_gamma.reshape(1, D), ln_beta.reshape(1, D))


@jax.jit
def kernel(x, emb_table, W, b, bn_gamma, bn_beta, ln_gamma, ln_beta):
    x3 = x.astype(jnp.int32).reshape(NW, CHUNKS, CHUNK_IDX)
    pooled = _pool_sc(x3, emb_table)
    return _dense_tc(pooled, W, b, bn_gamma, bn_beta, ln_gamma, ln_beta)
